# baseline (speedup 1.0000x reference)
import functools

import jax
import jax.numpy as jnp
from jax import lax
from jax.experimental import pallas as pl
from jax.experimental.pallas import tpu as pltpu
from jax.experimental.pallas import tpu_sc as plsc

N = 10000
NUM_EDGES = 320000
D = 128

NC = 2
NS = 16
NW = NC * NS
EDGES_PER_TILE = NUM_EDGES // NW
CHUNK = 50
NCHUNK = EDGES_PER_TILE // CHUNK
BLK = 8
NBLKI = NCHUNK // BLK
NP = 10240
DEG_STRIPE = NP // NS
ROW_STRIPE = NP // NS
HCHUNK = 80
HNCHUNK = EDGES_PER_TILE // HCHUNK

_MESH = plsc.VectorSubcoreMesh(core_axis_name="c", subcore_axis_name="s")


@functools.partial(
    pl.kernel,
    out_type=jax.ShapeDtypeStruct((NC, NP), jnp.float32),
    mesh=_MESH,
    scratch_types=[
        pltpu.VMEM((HNCHUNK, HCHUNK), jnp.int32),
        pltpu.VMEM((HCHUNK,), jnp.float32),
        pltpu.VMEM((DEG_STRIPE,), jnp.float32),
        pltpu.VMEM_SHARED((NP,), jnp.float32),
        pltpu.SemaphoreType.DMA,
    ],
)
def _degree_kernel(dsth_hbm, deg_out, dstv, ones, zbuf, deg_sp, hsem):
    c = lax.axis_index("c")
    s = lax.axis_index("s")
    wid = s * NC + c
    pltpu.sync_copy(dsth_hbm.at[wid], dstv)
    for i in range(DEG_STRIPE // 16):
        zbuf[pl.ds(i * 16, 16)] = jnp.zeros((16,), jnp.float32)
    for i in range(HCHUNK // 16):
        ones[pl.ds(i * 16, 16)] = jnp.ones((16,), jnp.float32)
    pltpu.sync_copy(zbuf, deg_sp.at[pl.ds(s * DEG_STRIPE, DEG_STRIPE)])
    plsc.subcore_barrier()

    def body(j, carry):
        pltpu.async_copy(ones, deg_sp.at[dstv.at[j]], hsem, add=True)
        return carry

    lax.fori_loop(0, HNCHUNK, body, 0)

    def drain(j, carry):
        pltpu.make_async_copy(ones, deg_sp.at[dstv.at[j]], hsem).wait()
        return carry

    lax.fori_loop(0, HNCHUNK, drain, 0)
    plsc.subcore_barrier()
    pltpu.sync_copy(deg_sp.at[pl.ds(s * DEG_STRIPE, DEG_STRIPE)],
                    deg_out.at[c, pl.ds(s * DEG_STRIPE, DEG_STRIPE)])


@functools.partial(
    pl.kernel,
    out_type=jax.ShapeDtypeStruct((NC, NP, D), jnp.float32),
    mesh=_MESH,
    scratch_types=[
        pltpu.VMEM_SHARED((NP, D), jnp.float32),
        pltpu.VMEM((BLK, CHUNK), jnp.int32),
        pltpu.VMEM((BLK, CHUNK), jnp.int32),
        pltpu.VMEM((CHUNK, D), jnp.float32),
        pltpu.VMEM((CHUNK, D), jnp.float32),
    ],
)
def _edge_kernel(src_hbm, dst_hbm, xs_hbm, acc_out, acc_sp, sblk, dblk,
                 bufa, bufb):
    c = lax.axis_index("c")
    s = lax.axis_index("s")
    wid = s * NC + c
    pltpu.sync_copy(xs_hbm.at[pl.ds(s * ROW_STRIPE, ROW_STRIPE)],
                    acc_sp.at[pl.ds(s * ROW_STRIPE, ROW_STRIPE)])
    plsc.subcore_barrier()

    def loop(gsem, ssem):
        def gath(r, buf):
            return pltpu.make_async_copy(xs_hbm.at[sblk.at[r]], buf, gsem)

        def scat(r, buf):
            return pltpu.make_async_copy(buf, acc_sp.at[dblk.at[r]], ssem)

        def body(b, carry):
            pltpu.sync_copy(src_hbm.at[wid * NBLKI + b], sblk)
            pltpu.sync_copy(dst_hbm.at[wid * NBLKI + b], dblk)
            for k in range(BLK // 2):
                r0, r1 = 2 * k, 2 * k + 1
                gath(r0, bufa).start()
                gath(r0, bufa).wait()
                scat(r0, bufa).start(add=True)
                gath(r1, bufb).start()
                gath(r1, bufb).wait()
                scat(r0, bufa).wait()
                scat(r1, bufb).start(add=True)
                scat(r1, bufb).wait()
            return carry

        lax.fori_loop(0, NBLKI, body, 0)

    pl.run_scoped(loop, pltpu.SemaphoreType.DMA(()), pltpu.SemaphoreType.DMA(()))
    plsc.subcore_barrier()
    pltpu.sync_copy(acc_sp.at[pl.ds(s * ROW_STRIPE, ROW_STRIPE)],
                    acc_out.at[c, pl.ds(s * ROW_STRIPE, ROW_STRIPE)])


def _scale_body(degcol_ref, x_ref, rcol_ref, xs_ref):
    r = lax.rsqrt(degcol_ref[...])
    rcol_ref[...] = jnp.concatenate(
        [r, jnp.ones((NP - N, 1), jnp.float32)], axis=0)
    xs_ref[...] = jnp.concatenate(
        [x_ref[...] * r, jnp.zeros((NP - N, D), jnp.float32)], axis=0)


_scale_call = pl.pallas_call(
    _scale_body,
    out_shape=[
        jax.ShapeDtypeStruct((NP, 1), jnp.float32),
        jax.ShapeDtypeStruct((NP, D), jnp.float32),
    ],
)


def _out_body(acc_ref, xs_ref, rcol_ref, w_ref, b_ref, o_ref):
    m = (acc_ref[0] + acc_ref[1] - xs_ref[...]) * rcol_ref[...]
    o_ref[...] = (jnp.dot(m, w_ref[...],
                          preferred_element_type=jnp.float32)[:N]
                  + b_ref[...])


_out_call = pl.pallas_call(
    _out_body,
    out_shape=jax.ShapeDtypeStruct((N, D), jnp.float32),
)


def kernel(V, E, X, W, b):
    src_r = E[0].reshape(NW * NBLKI, BLK, CHUNK)
    dst_r = E[1].reshape(NW * NBLKI, BLK, CHUNK)
    dst_h = E[1].reshape(NW, HNCHUNK, HCHUNK)
    deg2 = _degree_kernel(dst_h)
    degcol = (deg2[0, :N] + deg2[1, :N] + 1.0).reshape(N, 1)
    rcol, xs = _scale_call(degcol, X)
    acc2 = _edge_kernel(src_r, dst_r, xs)
    return _out_call(acc2, xs, rcol, W, b.reshape(1, D))


# speedup vs baseline: 1.3607x; 1.3607x over previous
import functools

import jax
import jax.numpy as jnp
from jax import lax
from jax.experimental import pallas as pl
from jax.experimental.pallas import tpu as pltpu
from jax.experimental.pallas import tpu_sc as plsc

N = 10000
NUM_EDGES = 320000
D = 128

NC = 2
NS = 16
NW = NC * NS
EDGES_PER_TILE = NUM_EDGES // NW
CHUNK = 125
NCHUNK = EDGES_PER_TILE // CHUNK
NP = 10240
DEG_STRIPE = NP // NS
ROW_STRIPE = NP // NS
HCHUNK = 80
HNCHUNK = EDGES_PER_TILE // HCHUNK

_MESH = plsc.VectorSubcoreMesh(core_axis_name="c", subcore_axis_name="s")


@functools.partial(
    pl.kernel,
    out_type=jax.ShapeDtypeStruct((NC, NP), jnp.float32),
    mesh=_MESH,
    scratch_types=[
        pltpu.VMEM((HNCHUNK, HCHUNK), jnp.int32),
        pltpu.VMEM((HCHUNK,), jnp.float32),
        pltpu.VMEM((DEG_STRIPE,), jnp.float32),
        pltpu.VMEM_SHARED((NP,), jnp.float32),
        pltpu.SemaphoreType.DMA,
    ],
)
def _degree_kernel(dsth_hbm, deg_out, dstv, ones, zbuf, deg_sp, hsem):
    c = lax.axis_index("c")
    s = lax.axis_index("s")
    wid = s * NC + c
    pltpu.sync_copy(dsth_hbm.at[wid], dstv)
    for i in range(DEG_STRIPE // 16):
        zbuf[pl.ds(i * 16, 16)] = jnp.zeros((16,), jnp.float32)
    for i in range(HCHUNK // 16):
        ones[pl.ds(i * 16, 16)] = jnp.ones((16,), jnp.float32)
    pltpu.sync_copy(zbuf, deg_sp.at[pl.ds(s * DEG_STRIPE, DEG_STRIPE)])
    plsc.subcore_barrier()

    def body(j, carry):
        pltpu.async_copy(ones, deg_sp.at[dstv.at[j]], hsem, add=True)
        return carry

    lax.fori_loop(0, HNCHUNK, body, 0)

    def drain(j, carry):
        pltpu.make_async_copy(ones, deg_sp.at[dstv.at[j]], hsem).wait()
        return carry

    lax.fori_loop(0, HNCHUNK, drain, 0)
    plsc.subcore_barrier()
    pltpu.sync_copy(deg_sp.at[pl.ds(s * DEG_STRIPE, DEG_STRIPE)],
                    deg_out.at[c, pl.ds(s * DEG_STRIPE, DEG_STRIPE)])


@functools.partial(
    pl.kernel,
    out_type=jax.ShapeDtypeStruct((NC, NP, D), jnp.float32),
    mesh=_MESH,
    scratch_types=[
        pltpu.VMEM_SHARED((NP, D), jnp.float32),
        pltpu.VMEM((NCHUNK, CHUNK), jnp.int32),
        pltpu.VMEM((NCHUNK, CHUNK), jnp.int32),
        pltpu.VMEM((CHUNK, D), jnp.float32),
    ],
)
def _edge_kernel(src_hbm, dst_hbm, xs_hbm, acc_out, acc_sp, srcv, dstv, buf):
    c = lax.axis_index("c")
    s = lax.axis_index("s")
    wid = s * NC + c
    pltpu.sync_copy(src_hbm.at[wid], srcv)
    pltpu.sync_copy(dst_hbm.at[wid], dstv)
    pltpu.sync_copy(xs_hbm.at[pl.ds(s * ROW_STRIPE, ROW_STRIPE)],
                    acc_sp.at[pl.ds(s * ROW_STRIPE, ROW_STRIPE)])
    plsc.subcore_barrier()

    def body(j, carry):
        pltpu.sync_copy(xs_hbm.at[srcv.at[j]], buf)
        pltpu.sync_copy(buf, acc_sp.at[dstv.at[j]], add=True)
        return carry

    lax.fori_loop(0, NCHUNK, body, 0)
    plsc.subcore_barrier()
    pltpu.sync_copy(acc_sp.at[pl.ds(s * ROW_STRIPE, ROW_STRIPE)],
                    acc_out.at[c, pl.ds(s * ROW_STRIPE, ROW_STRIPE)])


def _scale_body(degcol_ref, x_ref, rcol_ref, xs_ref):
    r = lax.rsqrt(degcol_ref[...])
    rcol_ref[...] = jnp.concatenate(
        [r, jnp.ones((NP - N, 1), jnp.float32)], axis=0)
    xs_ref[...] = jnp.concatenate(
        [x_ref[...] * r, jnp.zeros((NP - N, D), jnp.float32)], axis=0)


_scale_call = pl.pallas_call(
    _scale_body,
    out_shape=[
        jax.ShapeDtypeStruct((NP, 1), jnp.float32),
        jax.ShapeDtypeStruct((NP, D), jnp.float32),
    ],
)


def _out_body(acc_ref, xs_ref, rcol_ref, w_ref, b_ref, o_ref):
    m = (acc_ref[0] + acc_ref[1] - xs_ref[...]) * rcol_ref[...]
    o_ref[...] = (jnp.dot(m, w_ref[...],
                          preferred_element_type=jnp.float32)[:N]
                  + b_ref[...])


_out_call = pl.pallas_call(
    _out_body,
    out_shape=jax.ShapeDtypeStruct((N, D), jnp.float32),
)


def kernel(V, E, X, W, b):
    src_r = E[0].reshape(NW, NCHUNK, CHUNK)
    dst_r = E[1].reshape(NW, NCHUNK, CHUNK)
    dst_h = E[1].reshape(NW, HNCHUNK, HCHUNK)
    deg2 = _degree_kernel(dst_h)
    degcol = (deg2[0, :N] + deg2[1, :N] + 1.0).reshape(N, 1)
    rcol, xs = _scale_call(degcol, X)
    acc2 = _edge_kernel(src_r, dst_r, xs)
    return _out_call(acc2, xs, rcol, W, b.reshape(1, D))
